# baseline (speedup 1.0000x reference)
import functools

import jax
import jax.numpy as jnp
from jax import lax
from jax.experimental import pallas as pl
from jax.experimental.pallas import tpu as pltpu
from jax.experimental.pallas import tpu_sc as plsc

NUM_CORES = 2
NUM_SUBCORES = 16
LANES = 16
NW = NUM_CORES * NUM_SUBCORES
IDX_CHUNK = 128


def _sc_partials(label3, feat3, centers):
    _, ic, _ = label3.shape
    _, bpw, d = feat3.shape
    nvec = d // LANES
    mesh = plsc.VectorSubcoreMesh(core_axis_name="c", subcore_axis_name="s")

    @functools.partial(
        pl.kernel,
        mesh=mesh,
        out_type=jax.ShapeDtypeStruct((NW, LANES), jnp.float32),
        scratch_types=[
            pltpu.VMEM((ic, IDX_CHUNK), jnp.int32),
            pltpu.VMEM((bpw, d), jnp.float32),
            pltpu.VMEM((bpw, d), jnp.float32),
            pltpu.VMEM((LANES,), jnp.float32),
            pltpu.SemaphoreType.DMA,
        ],
        compiler_params=pltpu.CompilerParams(use_tc_tiling_on_sc=False),
    )
    def k(label_hbm, feat_hbm, centers_hbm, out_hbm, idx_v, feat_v, rows_v,
          part_v, sem):
        wid = lax.axis_index("s") * NUM_CORES + lax.axis_index("c")
        pltpu.sync_copy(label_hbm.at[wid], idx_v)
        gathers = [
            pltpu.async_copy(
                centers_hbm.at[idx_v.at[c]],
                rows_v.at[pl.ds(c * IDX_CHUNK, IDX_CHUNK)],
                sem,
            )
            for c in range(ic)
        ]
        pltpu.sync_copy(feat_hbm.at[wid], feat_v)
        for g in gathers:
            g.wait()

        def body(i, accs):
            new = []
            for j in range(nvec):
                f = feat_v[i, pl.ds(j * LANES, LANES)]
                c = rows_v[i, pl.ds(j * LANES, LANES)]
                diff = f - c
                new.append(accs[j] + diff * diff)
            return tuple(new)

        accs = lax.fori_loop(
            0, bpw, body,
            tuple(jnp.zeros((LANES,), jnp.float32) for _ in range(nvec)))
        total = accs[0]
        for j in range(1, nvec):
            total = total + accs[j]
        part_v[...] = total
        pltpu.sync_copy(part_v, out_hbm.at[wid])

    return k(label3, feat3, centers)


def _reduce_partials(partials, scale):
    def rk(p_ref, o_ref):
        o_ref[0, 0] = jnp.sum(p_ref[...]) * scale

    return pl.pallas_call(
        rk,
        out_shape=jax.ShapeDtypeStruct((1, 1), jnp.float32),
        out_specs=pl.BlockSpec(memory_space=pltpu.SMEM),
    )(partials)


def kernel(label, feat, centers):
    batch = feat.shape[0]
    feat = feat.reshape(batch, -1)
    d = feat.shape[1]
    label3 = label.astype(jnp.int32).reshape(NW, -1, IDX_CHUNK)
    feat3 = feat.reshape(NW, batch // NW, d)
    partials = _sc_partials(label3, feat3, centers)
    out = _reduce_partials(partials, 0.5 / batch)
    return out[0, 0]


# speedup vs baseline: 1.0021x; 1.0021x over previous
import functools

import jax
import jax.numpy as jnp
from jax import lax
from jax.experimental import pallas as pl
from jax.experimental.pallas import tpu as pltpu
from jax.experimental.pallas import tpu_sc as plsc

NUM_CORES = 2
NUM_SUBCORES = 16
LANES = 16
NW = NUM_CORES * NUM_SUBCORES
IDX_CHUNK = 128


def _sc_partials(label, feat, centers):
    b, d = feat.shape
    bpw = b // NW
    ic = bpw // IDX_CHUNK
    nvec = d // LANES
    mesh = plsc.VectorSubcoreMesh(core_axis_name="c", subcore_axis_name="s")

    @functools.partial(
        pl.kernel,
        mesh=mesh,
        out_type=jax.ShapeDtypeStruct((NW, LANES), jnp.float32),
        scratch_types=[
            pltpu.VMEM((bpw,), jnp.int32),
            pltpu.VMEM((bpw, d), jnp.float32),
            pltpu.VMEM((bpw, d), jnp.float32),
            pltpu.VMEM((LANES,), jnp.float32),
            pltpu.SemaphoreType.DMA,
        ],
        compiler_params=pltpu.CompilerParams(use_tc_tiling_on_sc=False),
    )
    def k(label_hbm, feat_hbm, centers_hbm, out_hbm, idx_v, feat_v, rows_v,
          part_v, sem):
        wid = lax.axis_index("s") * NUM_CORES + lax.axis_index("c")
        base = wid * bpw
        pltpu.sync_copy(label_hbm.at[pl.ds(base, bpw)], idx_v)
        gathers = [
            pltpu.async_copy(
                centers_hbm.at[idx_v.at[pl.ds(c * IDX_CHUNK, IDX_CHUNK)]],
                rows_v.at[pl.ds(c * IDX_CHUNK, IDX_CHUNK)],
                sem,
            )
            for c in range(ic)
        ]
        pltpu.sync_copy(feat_hbm.at[pl.ds(base, bpw)], feat_v)
        for g in gathers:
            g.wait()

        def body(i, accs):
            new = []
            for j in range(nvec):
                f = feat_v[i, pl.ds(j * LANES, LANES)]
                c = rows_v[i, pl.ds(j * LANES, LANES)]
                diff = f - c
                new.append(accs[j] + diff * diff)
            return tuple(new)

        accs = lax.fori_loop(
            0, bpw, body,
            tuple(jnp.zeros((LANES,), jnp.float32) for _ in range(nvec)))
        total = accs[0]
        for j in range(1, nvec):
            total = total + accs[j]
        part_v[...] = total
        pltpu.sync_copy(part_v, out_hbm.at[wid])

    return k(label, feat, centers)


def _reduce_partials(partials, scale):
    def rk(p_ref, o_ref):
        o_ref[0, 0] = jnp.sum(p_ref[...]) * scale

    return pl.pallas_call(
        rk,
        out_shape=jax.ShapeDtypeStruct((1, 1), jnp.float32),
        out_specs=pl.BlockSpec(memory_space=pltpu.SMEM),
    )(partials)


def kernel(label, feat, centers):
    batch = feat.shape[0]
    feat = feat.reshape(batch, -1)
    partials = _sc_partials(label.astype(jnp.int32), feat, centers)
    out = _reduce_partials(partials, 0.5 / batch)
    return out[0, 0]


# speedup vs baseline: 1.4819x; 1.4788x over previous
import functools

import jax
import jax.numpy as jnp
from jax import lax
from jax.experimental import pallas as pl
from jax.experimental.pallas import tpu as pltpu
from jax.experimental.pallas import tpu_sc as plsc

NUM_CORES = 2
NUM_SUBCORES = 16
LANES = 16
NW = NUM_CORES * NUM_SUBCORES
CHUNK = 25088


def _sc_partials(label, feat_t, centers_t, tail):
    d_dim, b = feat_t.shape
    _, v = centers_t.shape
    d_per_w = d_dim // NW
    main_w = (v // 128) * 128
    tail_w = v - main_w
    n_chunks = -(-main_w // CHUNK)
    offs = [c * CHUNK for c in range(n_chunks)]
    lens = [min(CHUNK, main_w - o) for o in offs]
    n_vec = b // LANES
    mesh = plsc.VectorSubcoreMesh(core_axis_name="c", subcore_axis_name="s")

    @functools.partial(
        pl.kernel,
        mesh=mesh,
        out_type=jax.ShapeDtypeStruct((NW, 128), jnp.float32),
        scratch_types=[
            pltpu.VMEM((b,), jnp.int32),
            pltpu.VMEM((b,), jnp.float32),
            pltpu.VMEM((CHUNK,), jnp.float32),
            pltpu.VMEM((CHUNK,), jnp.float32),
            pltpu.VMEM((d_dim * tail_w,), jnp.float32),
            pltpu.VMEM((128,), jnp.float32),
            pltpu.SemaphoreType.DMA,
            pltpu.SemaphoreType.DMA,
            pltpu.SemaphoreType.DMA,
            pltpu.SemaphoreType.DMA,
            pltpu.SemaphoreType.DMA,
        ],
        compiler_params=pltpu.CompilerParams(
            use_tc_tiling_on_sc=True, needs_layout_passes=False),
    )
    def k(label_hbm, feat_hbm, centers_hbm, tail_hbm, out_hbm, lab_v, frow_v,
          crow0_v, crow1_v, tail_v, part_v, lab_sem, frow_sem, crow0_sem,
          crow1_sem, tail_sem):
        wid = lax.axis_index("s") * NUM_CORES + lax.axis_index("c")
        crow_v = [crow0_v, crow1_v]
        crow_sem = [crow0_sem, crow1_sem]

        tasks = [(di, c) for di in range(d_per_w) for c in range(n_chunks)]

        def start_crow(t, buf):
            di, c = tasks[t]
            return pltpu.async_copy(
                centers_hbm.at[wid * d_per_w + di, pl.ds(offs[c], lens[c])],
                crow_v[buf].at[pl.ds(0, lens[c])],
                crow_sem[buf],
            )

        lab_cp = pltpu.async_copy(label_hbm, lab_v, lab_sem)
        tail_cp = pltpu.async_copy(tail_hbm, tail_v, tail_sem)
        pend = start_crow(0, 0)
        frow_cp = pltpu.async_copy(
            feat_hbm.at[wid * d_per_w], frow_v, frow_sem)
        lab_cp.wait()
        tail_cp.wait()

        acc = jnp.zeros((LANES,), jnp.float32)
        for t, (di, c) in enumerate(tasks):
            buf = t % 2
            pend.wait()
            if t + 1 < len(tasks):
                pend = start_crow(t + 1, 1 - buf)
            if c == 0:
                frow_cp.wait()
            off = offs[c]
            main_len = lens[c]
            is_last = c == n_chunks - 1
            width = main_len + (tail_w if is_last else 0)
            crow = crow_v[buf]
            tail_base = (wid * d_per_w + di) * tail_w - main_len

            def body(i, a, _off=off, _w=width, _ml=main_len, _crow=crow,
                     _last=is_last, _tb=tail_base):
                s = i * LANES
                lab = lab_v[pl.ds(s, LANES)]
                rel = lab - _off
                inr = (rel >= 0) & (rel < _w)
                if _last:
                    in_main = rel < _ml
                    safe = jnp.where(inr & in_main, rel, 0)
                    cmain = plsc.load_gather(_crow, [safe])
                    tidx = jnp.where(inr & (~in_main), rel + _tb, 0)
                    ctail = plsc.load_gather(tail_v, [tidx])
                    cval = jnp.where(in_main, cmain, ctail)
                else:
                    safe = jnp.where(inr, rel, 0)
                    cval = plsc.load_gather(_crow, [safe])
                fval = frow_v[pl.ds(s, LANES)]
                diff = fval - cval
                return a + jnp.where(inr, diff * diff, jnp.float32(0.0))

            acc = lax.fori_loop(0, n_vec, body, acc)
            if is_last and di + 1 < d_per_w:
                frow_cp = pltpu.async_copy(
                    feat_hbm.at[wid * d_per_w + di + 1], frow_v, frow_sem)

        zero = jnp.zeros((LANES,), jnp.float32)
        for j in range(128 // LANES):
            part_v[pl.ds(j * LANES, LANES)] = zero
        part_v[pl.ds(0, LANES)] = acc
        pltpu.sync_copy(part_v, out_hbm.at[wid])

    return k(label, feat_t, centers_t, tail)


def _reduce_partials(partials, scale):
    def rk(p_ref, o_ref):
        o_ref[0, 0] = jnp.sum(p_ref[...]) * scale

    return pl.pallas_call(
        rk,
        out_shape=jax.ShapeDtypeStruct((1, 1), jnp.float32),
        out_specs=pl.BlockSpec(memory_space=pltpu.SMEM),
    )(partials)


def kernel(label, feat, centers):
    batch = feat.shape[0]
    feat = feat.reshape(batch, -1)
    centers_t = centers.T
    v = centers_t.shape[1]
    main_w = (v // 128) * 128
    tail = centers_t[:, main_w:].reshape(-1)
    partials = _sc_partials(label.astype(jnp.int32), feat.T, centers_t, tail)
    out = _reduce_partials(partials, 0.5 / batch)
    return out[0, 0]


# speedup vs baseline: 2.3763x; 1.6036x over previous
import functools

import jax
import jax.numpy as jnp
from jax import lax
from jax.experimental import pallas as pl
from jax.experimental.pallas import tpu as pltpu
from jax.experimental.pallas import tpu_sc as plsc

NUM_CORES = 2
NUM_SUBCORES = 16
LANES = 16
NW = NUM_CORES * NUM_SUBCORES
CHUNK = 25088
FSEG = 4096


def _sc_partials(label, feat_t, centers_t, tailp):
    d_dim, b = feat_t.shape
    _, v = centers_t.shape
    d_per_w = d_dim // NW
    main_w = (v // 128) * 128
    row_pad = main_w + 128
    n_chunks = -(-main_w // CHUNK)
    offs = [c * CHUNK for c in range(n_chunks)]
    lens = [min(CHUNK, main_w - o) for o in offs]
    n_segs = b // FSEG
    n_vec = FSEG // LANES
    mesh = plsc.VectorSubcoreMesh(core_axis_name="c", subcore_axis_name="s")

    @functools.partial(
        pl.kernel,
        mesh=mesh,
        out_type=jax.ShapeDtypeStruct((NW, 128), jnp.float32),
        scratch_types=[
            pltpu.VMEM((b,), jnp.int32),
            pltpu.VMEM((FSEG,), jnp.float32),
            pltpu.VMEM((FSEG,), jnp.float32),
            pltpu.VMEM((row_pad,), jnp.float32),
            pltpu.VMEM((128,), jnp.float32),
            pltpu.SemaphoreType.DMA,
            pltpu.SemaphoreType.DMA,
            pltpu.SemaphoreType.DMA,
            pltpu.SemaphoreType.DMA,
        ],
        compiler_params=pltpu.CompilerParams(
            use_tc_tiling_on_sc=True, needs_layout_passes=False),
    )
    def k(label_hbm, feat_hbm, centers_hbm, tailp_hbm, out_hbm, lab_v,
          frow0_v, frow1_v, row_v, part_v, lab_sem, frow0_sem, frow1_sem,
          row_sem):
        wid = lax.axis_index("s") * NUM_CORES + lax.axis_index("c")
        frow_v = [frow0_v, frow1_v]
        frow_sem = [frow0_sem, frow1_sem]
        segs = [(di, q) for di in range(d_per_w) for q in range(n_segs)]

        def start_row(di):
            d = wid * d_per_w + di
            cps = [
                pltpu.async_copy(
                    centers_hbm.at[d, pl.ds(offs[c], lens[c])],
                    row_v.at[pl.ds(offs[c], lens[c])],
                    row_sem,
                )
                for c in range(n_chunks)
            ]
            cps.append(pltpu.async_copy(
                tailp_hbm.at[d], row_v.at[pl.ds(main_w, 128)], row_sem))
            return cps

        def start_frow(s, buf):
            di, q = segs[s]
            return pltpu.async_copy(
                feat_hbm.at[wid * d_per_w + di, pl.ds(q * FSEG, FSEG)],
                frow_v[buf], frow_sem[buf])

        lab_cp = pltpu.async_copy(label_hbm, lab_v, lab_sem)
        row_cps = start_row(0)
        fpend = start_frow(0, 0)
        lab_cp.wait()

        acc = jnp.zeros((LANES,), jnp.float32)
        for s, (di, q) in enumerate(segs):
            buf = s % 2
            if q == 0:
                for cp in row_cps:
                    cp.wait()
            fpend.wait()
            if s + 1 < len(segs):
                fpend = start_frow(s + 1, 1 - buf)
            frow = frow_v[buf]

            def body(i, a, _base=q * FSEG, _frow=frow):
                st = i * LANES
                lab = lab_v[pl.ds(_base + st, LANES)]
                cval = plsc.load_gather(row_v, [lab])
                fval = _frow[pl.ds(st, LANES)]
                diff = fval - cval
                return a + diff * diff

            acc = lax.fori_loop(0, n_vec, body, acc, unroll=4)
            if q == n_segs - 1 and di + 1 < d_per_w:
                row_cps = start_row(di + 1)

        zero = jnp.zeros((LANES,), jnp.float32)
        for j in range(128 // LANES):
            part_v[pl.ds(j * LANES, LANES)] = zero
        part_v[pl.ds(0, LANES)] = acc
        pltpu.sync_copy(part_v, out_hbm.at[wid])

    return k(label, feat_t, centers_t, tailp)


def _reduce_partials(partials, scale):
    def rk(p_ref, o_ref):
        o_ref[0, 0] = jnp.sum(p_ref[...]) * scale

    return pl.pallas_call(
        rk,
        out_shape=jax.ShapeDtypeStruct((1, 1), jnp.float32),
        out_specs=pl.BlockSpec(memory_space=pltpu.SMEM),
    )(partials)


def kernel(label, feat, centers):
    batch = feat.shape[0]
    feat = feat.reshape(batch, -1)
    centers_t = centers.T
    v = centers_t.shape[1]
    main_w = (v // 128) * 128
    tailp = jnp.pad(centers_t[:, main_w:], ((0, 0), (0, 128 - (v - main_w))))
    partials = _sc_partials(label.astype(jnp.int32), feat.T, centers_t, tailp)
    out = _reduce_partials(partials, 0.5 / batch)
    return out[0, 0]


# speedup vs baseline: 2.3837x; 1.0031x over previous
import functools

import jax
import jax.numpy as jnp
from jax import lax
from jax.experimental import pallas as pl
from jax.experimental.pallas import tpu as pltpu
from jax.experimental.pallas import tpu_sc as plsc

NUM_CORES = 2
NUM_SUBCORES = 16
LANES = 16
NW = NUM_CORES * NUM_SUBCORES
CHUNK = 25088
FSEG = 4096


def _sc_partials(label, feat_t, centers_t, tailp):
    d_dim, b = feat_t.shape
    _, v = centers_t.shape
    d_per_w = d_dim // NW
    main_w = (v // 128) * 128
    row_pad = main_w + 128
    n_chunks = -(-main_w // CHUNK)
    offs = [c * CHUNK for c in range(n_chunks)]
    lens = [min(CHUNK, main_w - o) for o in offs]
    n_segs = b // FSEG
    n_vec = FSEG // LANES
    mesh = plsc.VectorSubcoreMesh(core_axis_name="c", subcore_axis_name="s")

    @functools.partial(
        pl.kernel,
        mesh=mesh,
        out_type=jax.ShapeDtypeStruct((NW, 128), jnp.float32),
        scratch_types=[
            pltpu.VMEM((b,), jnp.int32),
            pltpu.VMEM((FSEG,), jnp.float32),
            pltpu.VMEM((FSEG,), jnp.float32),
            pltpu.VMEM((row_pad,), jnp.float32),
            pltpu.VMEM((128,), jnp.float32),
            pltpu.SemaphoreType.DMA,
            pltpu.SemaphoreType.DMA,
            pltpu.SemaphoreType.DMA,
            pltpu.SemaphoreType.DMA,
        ],
        compiler_params=pltpu.CompilerParams(
            use_tc_tiling_on_sc=True, needs_layout_passes=False),
    )
    def k(label_hbm, feat_hbm, centers_hbm, tailp_hbm, out_hbm, lab_v,
          frow0_v, frow1_v, row_v, part_v, lab_sem, frow0_sem, frow1_sem,
          row_sem):
        wid = lax.axis_index("s") * NUM_CORES + lax.axis_index("c")
        frow_v = [frow0_v, frow1_v]
        frow_sem = [frow0_sem, frow1_sem]
        segs = [(di, q) for di in range(d_per_w) for q in range(n_segs)]

        def start_row(di):
            d = wid * d_per_w + di
            cps = [
                pltpu.async_copy(
                    centers_hbm.at[d, pl.ds(offs[c], lens[c])],
                    row_v.at[pl.ds(offs[c], lens[c])],
                    row_sem,
                )
                for c in range(n_chunks)
            ]
            cps.append(pltpu.async_copy(
                tailp_hbm.at[d], row_v.at[pl.ds(main_w, 128)], row_sem))
            return cps

        def start_frow(s, buf):
            di, q = segs[s]
            return pltpu.async_copy(
                feat_hbm.at[wid * d_per_w + di, pl.ds(q * FSEG, FSEG)],
                frow_v[buf], frow_sem[buf])

        lab_cp = pltpu.async_copy(label_hbm, lab_v, lab_sem)
        row_cps = start_row(0)
        fpend = start_frow(0, 0)
        lab_cp.wait()

        acc = jnp.zeros((LANES,), jnp.float32)
        for s, (di, q) in enumerate(segs):
            buf = s % 2
            if q == 0:
                for cp in row_cps:
                    cp.wait()
            fpend.wait()
            if s + 1 < len(segs):
                fpend = start_frow(s + 1, 1 - buf)
            frow = frow_v[buf]

            def body(i, a, _base=q * FSEG, _frow=frow):
                st = i * LANES
                lab = lab_v[pl.ds(_base + st, LANES)]
                cval = plsc.load_gather(row_v, [lab])
                fval = _frow[pl.ds(st, LANES)]
                diff = fval - cval
                return a + diff * diff

            acc = lax.fori_loop(0, n_vec, body, acc, unroll=8)
            if q == n_segs - 1 and di + 1 < d_per_w:
                row_cps = start_row(di + 1)

        zero = jnp.zeros((LANES,), jnp.float32)
        for j in range(128 // LANES):
            part_v[pl.ds(j * LANES, LANES)] = zero
        part_v[pl.ds(0, LANES)] = acc
        pltpu.sync_copy(part_v, out_hbm.at[wid])

    return k(label, feat_t, centers_t, tailp)


def _reduce_partials(partials, scale):
    def rk(p_ref, o_ref):
        o_ref[0, 0] = jnp.sum(p_ref[...]) * scale

    return pl.pallas_call(
        rk,
        out_shape=jax.ShapeDtypeStruct((1, 1), jnp.float32),
        out_specs=pl.BlockSpec(memory_space=pltpu.SMEM),
    )(partials)


def kernel(label, feat, centers):
    batch = feat.shape[0]
    feat = feat.reshape(batch, -1)
    centers_t = centers.T
    v = centers_t.shape[1]
    main_w = (v // 128) * 128
    tailp = jnp.pad(centers_t[:, main_w:], ((0, 0), (0, 128 - (v - main_w))))
    partials = _sc_partials(label.astype(jnp.int32), feat.T, centers_t, tailp)
    out = _reduce_partials(partials, 0.5 / batch)
    return out[0, 0]


# speedup vs baseline: 2.5466x; 1.0684x over previous
import functools

import jax
import jax.numpy as jnp
from jax import lax
from jax.experimental import pallas as pl
from jax.experimental.pallas import tpu as pltpu
from jax.experimental.pallas import tpu_sc as plsc

NUM_CORES = 2
NUM_SUBCORES = 16
LANES = 16
NW = NUM_CORES * NUM_SUBCORES
CHUNK = 25088
FSEG = 4096


def _sc_partials(label, feat_t, centers_t, tailp):
    d_dim, b = feat_t.shape
    _, v = centers_t.shape
    d_per_w = d_dim // NW
    main_w = (v // 128) * 128
    row_pad = main_w + 128
    n_chunks = -(-main_w // CHUNK)
    offs = [c * CHUNK for c in range(n_chunks)]
    lens = [min(CHUNK, main_w - o) for o in offs]
    n_segs = b // FSEG
    n_vec = FSEG // LANES
    mesh = plsc.VectorSubcoreMesh(core_axis_name="c", subcore_axis_name="s")

    @functools.partial(
        pl.kernel,
        mesh=mesh,
        out_type=jax.ShapeDtypeStruct((NW, 128), jnp.float32),
        scratch_types=[
            pltpu.VMEM((b,), jnp.int32),
            pltpu.VMEM_SHARED((b,), jnp.int32),
            pltpu.VMEM((FSEG,), jnp.float32),
            pltpu.VMEM((FSEG,), jnp.float32),
            pltpu.VMEM((row_pad,), jnp.float32),
            pltpu.VMEM((128,), jnp.float32),
            pltpu.SemaphoreType.DMA,
            pltpu.SemaphoreType.DMA,
            pltpu.SemaphoreType.DMA,
            pltpu.SemaphoreType.DMA,
        ],
        compiler_params=pltpu.CompilerParams(
            use_tc_tiling_on_sc=True, needs_layout_passes=False),
    )
    def k(label_hbm, feat_hbm, centers_hbm, tailp_hbm, out_hbm, lab_v,
          labsh_v, frow0_v, frow1_v, row_v, part_v, lab_sem, frow0_sem,
          frow1_sem, row_sem):
        sid = lax.axis_index("s")
        wid = sid * NUM_CORES + lax.axis_index("c")
        lab_chunk = b // NUM_SUBCORES
        frow_v = [frow0_v, frow1_v]
        frow_sem = [frow0_sem, frow1_sem]
        segs = [(di, q) for di in range(d_per_w) for q in range(n_segs)]

        def start_row(di):
            d = wid * d_per_w + di
            cps = [
                pltpu.async_copy(
                    centers_hbm.at[d, pl.ds(offs[c], lens[c])],
                    row_v.at[pl.ds(offs[c], lens[c])],
                    row_sem,
                )
                for c in range(n_chunks)
            ]
            cps.append(pltpu.async_copy(
                tailp_hbm.at[d], row_v.at[pl.ds(main_w, 128)], row_sem))
            return cps

        def start_frow(s, buf):
            di, q = segs[s]
            return pltpu.async_copy(
                feat_hbm.at[wid * d_per_w + di, pl.ds(q * FSEG, FSEG)],
                frow_v[buf], frow_sem[buf])

        lab_cp = pltpu.async_copy(
            label_hbm.at[pl.ds(sid * lab_chunk, lab_chunk)],
            labsh_v.at[pl.ds(sid * lab_chunk, lab_chunk)], lab_sem)
        row_cps = start_row(0)
        fpend = start_frow(0, 0)
        lab_cp.wait()
        plsc.subcore_barrier()
        pltpu.sync_copy(labsh_v, lab_v)

        acc = jnp.zeros((LANES,), jnp.float32)
        for s, (di, q) in enumerate(segs):
            buf = s % 2
            if q == 0:
                for cp in row_cps:
                    cp.wait()
            fpend.wait()
            if s + 1 < len(segs):
                fpend = start_frow(s + 1, 1 - buf)
            frow = frow_v[buf]

            def body(i, a, _base=q * FSEG, _frow=frow):
                st = i * LANES
                lab = lab_v[pl.ds(_base + st, LANES)]
                cval = plsc.load_gather(row_v, [lab])
                fval = _frow[pl.ds(st, LANES)]
                diff = fval - cval
                return a + diff * diff

            acc = lax.fori_loop(0, n_vec, body, acc, unroll=8)
            if q == n_segs - 1 and di + 1 < d_per_w:
                row_cps = start_row(di + 1)

        zero = jnp.zeros((LANES,), jnp.float32)
        for j in range(128 // LANES):
            part_v[pl.ds(j * LANES, LANES)] = zero
        part_v[pl.ds(0, LANES)] = acc
        pltpu.sync_copy(part_v, out_hbm.at[wid])

    return k(label, feat_t, centers_t, tailp)


def _reduce_partials(partials, scale):
    def rk(p_ref, o_ref):
        o_ref[0, 0] = jnp.sum(p_ref[...]) * scale

    return pl.pallas_call(
        rk,
        out_shape=jax.ShapeDtypeStruct((1, 1), jnp.float32),
        out_specs=pl.BlockSpec(memory_space=pltpu.SMEM),
    )(partials)


def kernel(label, feat, centers):
    batch = feat.shape[0]
    feat = feat.reshape(batch, -1)
    centers_t = centers.T
    v = centers_t.shape[1]
    main_w = (v // 128) * 128
    tailp = jnp.pad(centers_t[:, main_w:], ((0, 0), (0, 128 - (v - main_w))))
    partials = _sc_partials(label.astype(jnp.int32), feat.T, centers_t, tailp)
    out = _reduce_partials(partials, 0.5 / batch)
    return out[0, 0]
